# baseline (speedup 1.0000x reference)
import dataclasses

import jax
import jax.numpy as jnp
from jax import lax
from jax.experimental import pallas as pl
from jax.experimental.pallas import tpu as pltpu
from jax.experimental.pallas import tpu_sc as plsc

B = 16384
D = 32
NC = 2
NS = 16
NW = NC * NS
BPW = B // NW
HALF = BPW // 2


def _gather_body(c0, c1, c2, c3, k0h, k1h, k2h, k3h, out_hbm,
                 kc0, kc1, kc2, kc3, kc4, kc5, kc6, kc7,
                 rows_a, rows_b, sem_a, sem_b, wsem_a, wsem_b):
    wid = lax.axis_index("s") * NC + lax.axis_index("c")
    base = wid * BPW
    tabs = ((c0, k0h), (c1, k1h), (c2, k2h), (c3, k3h))
    kall = (kc0, kc1, kc2, kc3, kc4, kc5, kc6, kc7)
    bufs = (rows_a, rows_b)
    sems = (sem_a, sem_b)
    wsems = (wsem_a, wsem_b)
    for t, (ch, kh) in enumerate(tabs):
        pltpu.sync_copy(kh.at[pl.ds(base, HALF)], kall[2 * t])
        pltpu.sync_copy(kh.at[pl.ds(base + HALF, HALF)], kall[2 * t + 1])
    copies = [None] * 8
    wcopies = [None] * 8
    copies[0] = pltpu.async_copy(tabs[0][0].at[kall[0]], bufs[0], sems[0])
    for i in range(8):
        h, t = divmod(i, 4)
        if i + 1 < 8:
            hn, tn = divmod(i + 1, 4)
            if i >= 1:
                wcopies[i - 1].wait()
            copies[i + 1] = pltpu.async_copy(
                tabs[tn][0].at[kall[2 * tn + hn]], bufs[(i + 1) % 2],
                sems[(i + 1) % 2],
            )
        copies[i].wait()
        wcopies[i] = pltpu.async_copy(
            bufs[i % 2],
            out_hbm.at[pl.ds(base + h * HALF, HALF), pl.ds(t * 4 * D, 4 * D)],
            wsems[i % 2],
        )
    wcopies[6].wait()
    wcopies[7].wait()


def _detile_body(i0, i1, i2, i3, out_ref):
    out_ref[...] = jnp.concatenate(
        [i0[...], i1[...], i2[...], i3[...]], axis=0
    ).T


KB = 8192


def _chunked(t):
    v = t.shape[0]
    grid = (v + 4 * KB - 1) // (4 * KB)
    v4 = grid * KB
    vblk = (v + KB - 1) // KB
    tt = t.T
    out = pl.pallas_call(
        _detile_body,
        grid=(grid,),
        in_specs=[
            pl.BlockSpec(
                (D, KB),
                lambda j, q=q, g=grid, m=vblk - 1: (0, jnp.minimum(q * g + j, m)),
            )
            for q in range(4)
        ],
        out_specs=pl.BlockSpec((KB, 4 * D), lambda j: (j, 0)),
        out_shape=jax.ShapeDtypeStruct((v4, 4 * D), jnp.float32),
    )(tt, tt, tt, tt)
    return out, v4


RB = 512


def _select_body(raw_ref, q_ref, out_ref):
    raw = raw_ref[...]
    qb = q_ref[...]
    parts = []
    for t in range(4):
        qt = qb[:, t:t + 1]
        acc = None
        for p in range(4):
            seg = raw[:, 128 * t + 32 * p:128 * t + 32 * p + 32]
            v = jnp.where(qt == p, seg, 0.0)
            acc = v if acc is None else acc + v
        parts.append(acc)
    out_ref[...] = jnp.concatenate(parts, axis=1)


def _select(raw, qstack):
    return pl.pallas_call(
        _select_body,
        grid=(B // RB,),
        in_specs=[
            pl.BlockSpec((RB, 16 * D), lambda j: (j, 0)),
            pl.BlockSpec((RB, 4), lambda j: (j, 0)),
        ],
        out_specs=pl.BlockSpec((RB, 4 * D), lambda j: (j, 0)),
        out_shape=jax.ShapeDtypeStruct((B, 4 * D), jnp.float32),
    )(raw, qstack)


def kernel(room_id, hotel, room_type, room_name,
           room_table, hotel_table, room_type_table, room_name_table):
    mesh = plsc.VectorSubcoreMesh(core_axis_name="c", subcore_axis_name="s")
    cp = pltpu.CompilerParams()
    if "needs_layout_passes" in pltpu.CompilerParams.__dataclass_fields__:
        cp = dataclasses.replace(cp, needs_layout_passes=False)
    gather = pl.kernel(
        _gather_body,
        out_type=jax.ShapeDtypeStruct((B, 16 * D), jnp.float32),
        mesh=mesh,
        compiler_params=cp,
        scratch_types=(
            [pltpu.VMEM((HALF,), jnp.int32) for _ in range(8)]
            + [pltpu.VMEM((HALF, 4 * D), jnp.float32) for _ in range(2)]
            + [pltpu.SemaphoreType.DMA for _ in range(4)]
        ),
    )
    tables = (room_table, hotel_table, room_type_table, room_name_table)
    chunked = [_chunked(t) for t in tables]
    chunks = [c for c, _ in chunked]
    idxs = [i.astype(jnp.int32)
            for i in (room_id, hotel, room_type, room_name)]
    ks = [i % v4 for i, (_, v4) in zip(idxs, chunked)]
    qs = [i // v4 for i, (_, v4) in zip(idxs, chunked)]
    raw = gather(*chunks, *ks)
    return _select(raw, jnp.stack(qs, axis=1))


# speedup vs baseline: 1.2752x; 1.2752x over previous
import dataclasses

import jax
import jax.numpy as jnp
from jax import lax
from jax.experimental import pallas as pl
from jax.experimental.pallas import tpu as pltpu
from jax.experimental.pallas import tpu_sc as plsc

B = 16384
D = 32
NC = 2
NS = 16
NW = NC * NS
BPW = B // NW
HALF = BPW // 2
NLANE = 16


def _make_gather_body(ntab):
    def body(*args):
        chs = args[:ntab]
        khs = args[ntab:2 * ntab]
        qhs = args[2 * ntab:3 * ntab]
        out_hbm = args[3 * ntab]
        sc = args[3 * ntab + 1:]
        kall = sc[:2 * ntab]
        qall = sc[2 * ntab:3 * ntab]
        rows_a, rows_b, stage_v, sem_a, sem_b = sc[3 * ntab:]
        wid = lax.axis_index("s") * NC + lax.axis_index("c")
        base = wid * BPW
        iota = lax.broadcasted_iota(jnp.int32, (NLANE,), 0)
        zero = jnp.zeros((NLANE,), jnp.int32)
        bufs = (rows_a, rows_b)
        sems = (sem_a, sem_b)
        for t in range(ntab):
            pltpu.sync_copy(khs[t].at[pl.ds(base, HALF)], kall[2 * t])
            pltpu.sync_copy(khs[t].at[pl.ds(base + HALF, HALF)],
                            kall[2 * t + 1])
            pltpu.sync_copy(qhs[t].at[pl.ds(base, BPW)], qall[t])
        n = 2 * ntab
        copies = [None] * n
        copies[0] = pltpu.async_copy(chs[0].at[kall[0]], bufs[0], sems[0])
        for i in range(n):
            h, t = divmod(i, ntab)
            if i + 1 < n:
                hn, tn = divmod(i + 1, ntab)
                copies[i + 1] = pltpu.async_copy(
                    chs[tn].at[kall[2 * tn + hn]], bufs[(i + 1) % 2],
                    sems[(i + 1) % 2],
                )
            copies[i].wait()
            rows_v = bufs[i % 2]
            qv = qall[t]

            @pl.loop(0, HALF // NLANE)
            def _(g):
                ridx = iota + g * NLANE
                q16 = qv[pl.ds(h * HALF + g * NLANE, NLANE)]
                cbase = q16 * D
                for c in range(D):
                    val = plsc.load_gather(rows_v, [ridx, cbase + c])
                    plsc.store_scatter(stage_v, [ridx, zero + (t * D + c)],
                                       val)

            if t == ntab - 1:
                pltpu.sync_copy(
                    stage_v, out_hbm.at[pl.ds(base + h * HALF, HALF)]
                )

    return body


def _gather(chunks, ks, qs):
    ntab = len(chunks)
    mesh = plsc.VectorSubcoreMesh(core_axis_name="c", subcore_axis_name="s")
    cp = pltpu.CompilerParams()
    if "needs_layout_passes" in pltpu.CompilerParams.__dataclass_fields__:
        cp = dataclasses.replace(cp, needs_layout_passes=False)
    fn = pl.kernel(
        _make_gather_body(ntab),
        out_type=jax.ShapeDtypeStruct((B, ntab * D), jnp.float32),
        mesh=mesh,
        compiler_params=cp,
        scratch_types=(
            [pltpu.VMEM((HALF,), jnp.int32) for _ in range(2 * ntab)]
            + [pltpu.VMEM((BPW,), jnp.int32) for _ in range(ntab)]
            + [pltpu.VMEM((HALF, 4 * D), jnp.float32) for _ in range(2)]
            + [pltpu.VMEM((HALF, ntab * D), jnp.float32)]
            + [pltpu.SemaphoreType.DMA, pltpu.SemaphoreType.DMA]
        ),
    )
    return fn(*chunks, *ks, *qs)


def _detile_body(i0, i1, i2, i3, out_ref):
    out_ref[...] = jnp.concatenate(
        [i0[...], i1[...], i2[...], i3[...]], axis=0
    ).T


KB = 8192


def _chunked(t):
    v = t.shape[0]
    grid = (v + 4 * KB - 1) // (4 * KB)
    v4 = grid * KB
    vblk = (v + KB - 1) // KB
    tt = t.T
    out = pl.pallas_call(
        _detile_body,
        grid=(grid,),
        in_specs=[
            pl.BlockSpec(
                (D, KB),
                lambda j, q=q, g=grid, m=vblk - 1: (0, jnp.minimum(q * g + j, m)),
            )
            for q in range(4)
        ],
        out_specs=pl.BlockSpec((KB, 4 * D), lambda j: (j, 0)),
        out_shape=jax.ShapeDtypeStruct((v4, 4 * D), jnp.float32),
    )(tt, tt, tt, tt)
    return out, v4


def kernel(room_id, hotel, room_type, room_name,
           room_table, hotel_table, room_type_table, room_name_table):
    idxs = [i.astype(jnp.int32)
            for i in (room_id, hotel, room_type, room_name)]
    small = [_chunked(t) for t in
             (hotel_table, room_type_table, room_name_table)]
    ks_s = [i % v4 for i, (_, v4) in zip(idxs[1:], small)]
    qs_s = [i // v4 for i, (_, v4) in zip(idxs[1:], small)]
    out_small = _gather([c for c, _ in small], ks_s, qs_s)

    room_chunks, room_v4 = _chunked(room_table)
    out_room = _gather([room_chunks], [idxs[0] % room_v4],
                       [idxs[0] // room_v4])
    return jnp.concatenate([out_room, out_small], axis=1)


# speedup vs baseline: 1.3099x; 1.0273x over previous
import dataclasses

import jax
import jax.numpy as jnp
from jax import lax
from jax.experimental import pallas as pl
from jax.experimental.pallas import tpu as pltpu
from jax.experimental.pallas import tpu_sc as plsc

B = 16384
D = 32
NC = 2
NS = 16
NW = NC * NS
BPW = B // NW
HALF = BPW // 2
NLANE = 16


def _make_gather_body(ntab):
    def body(*args):
        chs = args[:ntab]
        khs = args[ntab:2 * ntab]
        qhs = args[2 * ntab:3 * ntab]
        out_hbm = args[3 * ntab]
        sc = args[3 * ntab + 1:]
        kall = sc[:2 * ntab]
        qall = sc[2 * ntab:3 * ntab]
        rows_a, rows_b, stage_v, sem_a, sem_b = sc[3 * ntab:]
        wid = lax.axis_index("s") * NC + lax.axis_index("c")
        base = wid * BPW
        iota = lax.broadcasted_iota(jnp.int32, (NLANE,), 0)
        zero = jnp.zeros((NLANE,), jnp.int32)
        bufs = (rows_a, rows_b)
        sems = (sem_a, sem_b)
        for t in range(ntab):
            pltpu.sync_copy(khs[t].at[pl.ds(base, HALF)], kall[2 * t])
            pltpu.sync_copy(khs[t].at[pl.ds(base + HALF, HALF)],
                            kall[2 * t + 1])
        n = 2 * ntab
        copies = [None] * n
        copies[0] = pltpu.async_copy(chs[0].at[kall[0]], bufs[0], sems[0])
        for t in range(ntab):
            pltpu.sync_copy(qhs[t].at[pl.ds(base, BPW)], qall[t])
        for i in range(n):
            h, t = divmod(i, ntab)
            if i + 1 < n:
                hn, tn = divmod(i + 1, ntab)
                copies[i + 1] = pltpu.async_copy(
                    chs[tn].at[kall[2 * tn + hn]], bufs[(i + 1) % 2],
                    sems[(i + 1) % 2],
                )
            copies[i].wait()
            rows_v = bufs[i % 2]
            qv = qall[t]

            @pl.loop(0, HALF // NLANE)
            def _(g):
                ridx = iota + g * NLANE
                q16 = qv[pl.ds(h * HALF + g * NLANE, NLANE)]
                cbase = q16 * D
                for c in range(D):
                    val = plsc.load_gather(rows_v, [ridx, cbase + c])
                    plsc.store_scatter(stage_v, [ridx, zero + (t * D + c)],
                                       val)

            if t == ntab - 1:
                pltpu.sync_copy(
                    stage_v, out_hbm.at[pl.ds(base + h * HALF, HALF)]
                )

    return body


def _gather(chunks, ks, qs):
    ntab = len(chunks)
    mesh = plsc.VectorSubcoreMesh(core_axis_name="c", subcore_axis_name="s")
    cp = pltpu.CompilerParams()
    if "needs_layout_passes" in pltpu.CompilerParams.__dataclass_fields__:
        cp = dataclasses.replace(cp, needs_layout_passes=False)
    fn = pl.kernel(
        _make_gather_body(ntab),
        out_type=jax.ShapeDtypeStruct((B, ntab * D), jnp.float32),
        mesh=mesh,
        compiler_params=cp,
        scratch_types=(
            [pltpu.VMEM((HALF,), jnp.int32) for _ in range(2 * ntab)]
            + [pltpu.VMEM((BPW,), jnp.int32) for _ in range(ntab)]
            + [pltpu.VMEM((HALF, 4 * D), jnp.float32) for _ in range(2)]
            + [pltpu.VMEM((HALF, ntab * D), jnp.float32)]
            + [pltpu.SemaphoreType.DMA, pltpu.SemaphoreType.DMA]
        ),
    )
    return fn(*chunks, *ks, *qs)


def _detile_body(i0, i1, i2, i3, out_ref):
    out_ref[...] = jnp.concatenate(
        [i0[...], i1[...], i2[...], i3[...]], axis=0
    ).T


def _chunked(t):
    v = t.shape[0]
    kb = 16384 if v > 500000 else (512 if v < 4096 else 8192)
    grid = (v + 4 * kb - 1) // (4 * kb)
    v4 = grid * kb
    vblk = (v + kb - 1) // kb
    tt = t.T
    out = pl.pallas_call(
        _detile_body,
        grid=(grid,),
        in_specs=[
            pl.BlockSpec(
                (D, kb),
                lambda j, q=q, g=grid, m=vblk - 1: (0, jnp.minimum(q * g + j, m)),
            )
            for q in range(4)
        ],
        out_specs=pl.BlockSpec((kb, 4 * D), lambda j: (j, 0)),
        out_shape=jax.ShapeDtypeStruct((v4, 4 * D), jnp.float32),
    )(tt, tt, tt, tt)
    return out, v4


def kernel(room_id, hotel, room_type, room_name,
           room_table, hotel_table, room_type_table, room_name_table):
    idxs = [i.astype(jnp.int32)
            for i in (room_id, hotel, room_type, room_name)]
    small = [_chunked(t) for t in
             (hotel_table, room_type_table, room_name_table)]
    ks_s = [i % v4 for i, (_, v4) in zip(idxs[1:], small)]
    qs_s = [i // v4 for i, (_, v4) in zip(idxs[1:], small)]
    out_small = _gather([c for c, _ in small], ks_s, qs_s)

    room_chunks, room_v4 = _chunked(room_table)
    out_room = _gather([room_chunks], [idxs[0] % room_v4],
                       [idxs[0] // room_v4])
    return jnp.concatenate([out_room, out_small], axis=1)


# speedup vs baseline: 1.3147x; 1.0037x over previous
import dataclasses

import jax
import jax.numpy as jnp
from jax import lax
from jax.experimental import pallas as pl
from jax.experimental.pallas import tpu as pltpu
from jax.experimental.pallas import tpu_sc as plsc

B = 16384
D = 32
NC = 2
NS = 16
NW = NC * NS
BPW = B // NW
HALF = BPW // 2
QTR = BPW // 4
NLANE = 16


def _make_gather_body(ntab):
    def body(*args):
        chs = args[:ntab]
        khs = args[ntab:2 * ntab]
        qhs = args[2 * ntab:3 * ntab]
        out_hbm = args[3 * ntab]
        sc = args[3 * ntab + 1:]
        kall = sc[:4 * ntab]
        qall = sc[4 * ntab:5 * ntab]
        rows_a, rows_b, stage_v, sem_a, sem_b = sc[5 * ntab:]
        wid = lax.axis_index("s") * NC + lax.axis_index("c")
        base = wid * BPW
        iota = lax.broadcasted_iota(jnp.int32, (NLANE,), 0)
        zero = jnp.zeros((NLANE,), jnp.int32)
        bufs = (rows_a, rows_b)
        sems = (sem_a, sem_b)
        for t in range(ntab):
            for v in range(4):
                pltpu.sync_copy(khs[t].at[pl.ds(base + v * QTR, QTR)],
                                kall[4 * t + v])
        n = 4 * ntab
        copies = [None] * n
        copies[0] = pltpu.async_copy(chs[0].at[kall[0]], bufs[0], sems[0])
        for t in range(ntab):
            pltpu.sync_copy(qhs[t].at[pl.ds(base, BPW)], qall[t])
        for i in range(n):
            v, t = divmod(i, ntab)
            if i + 1 < n:
                vn, tn = divmod(i + 1, ntab)
                copies[i + 1] = pltpu.async_copy(
                    chs[tn].at[kall[4 * tn + vn]], bufs[(i + 1) % 2],
                    sems[(i + 1) % 2],
                )
            copies[i].wait()
            rows_v = bufs[i % 2]
            qv = qall[t]

            @pl.loop(0, QTR // NLANE)
            def _(g):
                ridx = iota + g * NLANE
                q16 = qv[pl.ds(v * QTR + g * NLANE, NLANE)]
                cbase = q16 * D
                for c in range(D):
                    val = plsc.load_gather(rows_v, [ridx, cbase + c])
                    plsc.store_scatter(stage_v, [ridx, zero + (t * D + c)],
                                       val)

            if t == ntab - 1:
                pltpu.sync_copy(
                    stage_v, out_hbm.at[pl.ds(base + v * QTR, QTR)]
                )

    return body


def _gather(chunks, ks, qs):
    ntab = len(chunks)
    mesh = plsc.VectorSubcoreMesh(core_axis_name="c", subcore_axis_name="s")
    cp = pltpu.CompilerParams()
    if "needs_layout_passes" in pltpu.CompilerParams.__dataclass_fields__:
        cp = dataclasses.replace(cp, needs_layout_passes=False)
    fn = pl.kernel(
        _make_gather_body(ntab),
        out_type=jax.ShapeDtypeStruct((B, ntab * D), jnp.float32),
        mesh=mesh,
        compiler_params=cp,
        scratch_types=(
            [pltpu.VMEM((QTR,), jnp.int32) for _ in range(4 * ntab)]
            + [pltpu.VMEM((BPW,), jnp.int32) for _ in range(ntab)]
            + [pltpu.VMEM((QTR, 4 * D), jnp.float32) for _ in range(2)]
            + [pltpu.VMEM((QTR, ntab * D), jnp.float32)]
            + [pltpu.SemaphoreType.DMA, pltpu.SemaphoreType.DMA]
        ),
    )
    return fn(*chunks, *ks, *qs)


def _detile_body(i0, i1, i2, i3, out_ref):
    out_ref[...] = jnp.concatenate(
        [i0[...], i1[...], i2[...], i3[...]], axis=0
    ).T


def _chunked(t):
    v = t.shape[0]
    kb = 16384 if v > 500000 else (512 if v < 4096 else 8192)
    grid = (v + 4 * kb - 1) // (4 * kb)
    v4 = grid * kb
    vblk = (v + kb - 1) // kb
    tt = t.T
    out = pl.pallas_call(
        _detile_body,
        grid=(grid,),
        in_specs=[
            pl.BlockSpec(
                (D, kb),
                lambda j, q=q, g=grid, m=vblk - 1: (0, jnp.minimum(q * g + j, m)),
            )
            for q in range(4)
        ],
        out_specs=pl.BlockSpec((kb, 4 * D), lambda j: (j, 0)),
        out_shape=jax.ShapeDtypeStruct((v4, 4 * D), jnp.float32),
    )(tt, tt, tt, tt)
    return out, v4


def kernel(room_id, hotel, room_type, room_name,
           room_table, hotel_table, room_type_table, room_name_table):
    idxs = [i.astype(jnp.int32)
            for i in (room_id, hotel, room_type, room_name)]
    small = [_chunked(t) for t in
             (hotel_table, room_type_table, room_name_table)]
    ks_s = [i % v4 for i, (_, v4) in zip(idxs[1:], small)]
    qs_s = [i // v4 for i, (_, v4) in zip(idxs[1:], small)]
    out_small = _gather([c for c, _ in small], ks_s, qs_s)

    room_chunks, room_v4 = _chunked(room_table)
    out_room = _gather([room_chunks], [idxs[0] % room_v4],
                       [idxs[0] // room_v4])
    return jnp.concatenate([out_room, out_small], axis=1)
